# baseline (speedup 1.0000x reference)
import jax
import jax.numpy as jnp
from jax import lax
from jax.experimental import pallas as pl
from jax.experimental.pallas import tpu as pltpu
from jax.experimental.pallas import tpu_sc as plsc

N = 10000
E = 320000
D = 128
H = 128
O = 128

NC = 2
NS = 16
NW = NC * NS
MB = 96
NB = 108
CH = 6
NCHUNK = NB // CH
EPW = NB * MB
E_PAD = NW * EPW
N_PAD = 10112
ROWS_PER_TILE = N_PAD // NS


def _proj_body(x_ref, ws_ref, wd_ref, as_ref, ad_ref, h_ref, asr_ref, adr_ref):
    x = x_ref[...]
    h = jnp.dot(x, ws_ref[...], preferred_element_type=jnp.float32)
    h_ref[...] = h
    a_s = jnp.dot(h, as_ref[...].T, preferred_element_type=jnp.float32)
    wd_v = jnp.dot(wd_ref[...], ad_ref[...].T, preferred_element_type=jnp.float32)
    a_d = jnp.dot(x, wd_v, preferred_element_type=jnp.float32)
    asr_ref[...] = a_s.reshape(asr_ref.shape)
    adr_ref[...] = a_d.reshape(adr_ref.shape)


def _projections(x, W_src, W_dst, att_src, att_dst):
    nb = 25
    bs = N // nb
    h, a_s, a_d = pl.pallas_call(
        _proj_body,
        grid=(nb,),
        in_specs=[
            pl.BlockSpec((bs, D), lambda i: (i, 0)),
            pl.BlockSpec((D, H), lambda i: (0, 0)),
            pl.BlockSpec((D, H), lambda i: (0, 0)),
            pl.BlockSpec((1, H), lambda i: (0, 0)),
            pl.BlockSpec((1, H), lambda i: (0, 0)),
        ],
        out_specs=[
            pl.BlockSpec((bs, H), lambda i: (i, 0)),
            pl.BlockSpec((1, 1, bs), lambda i: (i, 0, 0)),
            pl.BlockSpec((1, 1, bs), lambda i: (i, 0, 0)),
        ],
        out_shape=[
            jax.ShapeDtypeStruct((N, H), jnp.float32),
            jax.ShapeDtypeStruct((nb, 1, bs), jnp.float32),
            jax.ShapeDtypeStruct((nb, 1, bs), jnp.float32),
        ],
    )(x, W_src, W_dst, att_src.reshape(1, H), att_dst.reshape(1, H))
    return h, a_s.reshape(N), a_d.reshape(N)


def _edge_body(src_hbm, dst_hbm, asrc_hbm, adst_hbm, h_hbm,
               numer_out, denom_out,
               asrc_v, adst_v,
               csrc, cdst,
               ex0, ex1, rows0, rows1,
               numer_sh, denom_sh,
               g0, g1, s0, s1, csem):
    cid = lax.axis_index("c")
    sid = lax.axis_index("s")
    wid = sid * NC + cid
    ex_bufs = [ex0, ex1]
    rows_bufs = [rows0, rows1]
    gsems = [g0, g1]
    ssems = [s0, s1]
    ebase = wid * EPW

    zv = jnp.zeros((16,), jnp.float32)

    def zero_rows(r, _):
        for c in range(H // 16):
            rows0[r, pl.ds(c * 16, 16)] = zv
        return 0

    lax.fori_loop(0, MB, zero_rows, 0)
    for g in range(MB // 16):
        ex0[pl.ds(g * 16, 16)] = zv
    tbase = sid * ROWS_PER_TILE
    for j in range(ROWS_PER_TILE // MB):
        pltpu.sync_copy(rows0, numer_sh.at[pl.ds(tbase + j * MB, MB), :])
        pltpu.sync_copy(ex0, denom_sh.at[pl.ds(tbase + j * MB, MB)])
    rem = ROWS_PER_TILE % MB
    rbase = tbase + (ROWS_PER_TILE // MB) * MB
    pltpu.sync_copy(rows0.at[pl.ds(0, rem), :],
                    numer_sh.at[pl.ds(rbase, rem), :])
    pltpu.sync_copy(ex0.at[pl.ds(0, rem)], denom_sh.at[pl.ds(rbase, rem)])

    pltpu.sync_copy(asrc_hbm, asrc_v)
    pltpu.sync_copy(adst_hbm, adst_v)

    plsc.subcore_barrier()

    def chunk_row(b):
        return ((b // CH) % 2) * 8 + b % CH

    def chunk_slot(c):
        return pl.multiple_of((c % 2) * 8, 8)

    def issue_chunk(c):
        sl = chunk_slot(c)
        pltpu.async_copy(src_hbm.at[wid, c], csrc.at[pl.ds(sl, CH), :], csem)
        pltpu.async_copy(dst_hbm.at[wid, c], cdst.at[pl.ds(sl, CH), :], csem)

    def wait_chunk(c):
        sl = chunk_slot(c)
        pltpu.make_async_copy(src_hbm.at[wid, c], csrc.at[pl.ds(sl, CH), :],
                              csem).wait()
        pltpu.make_async_copy(dst_hbm.at[wid, c], cdst.at[pl.ds(sl, CH), :],
                              csem).wait()

    def compute_ex(b, m):
        row = chunk_row(b)
        ev = ex_bufs[m]
        for g in range(MB // 16):
            si = csrc[row, pl.ds(g * 16, 16)]
            di = cdst[row, pl.ds(g * 16, 16)]
            av = plsc.load_gather(asrc_v, [si])
            bv = plsc.load_gather(adst_v, [di])
            al = av + bv
            al = jnp.where(al >= 0.0, al, 0.2 * al)
            ev[pl.ds(g * 16, 16)] = jnp.exp(al)

    def start_gather(b, k):
        row = chunk_row(b)
        pltpu.async_copy(h_hbm.at[csrc.at[row]], rows_bufs[k], gsems[k])

    def wait_gather(b, k):
        row = chunk_row(b)
        pltpu.make_async_copy(h_hbm.at[csrc.at[row]], rows_bufs[k],
                              gsems[k]).wait()

    def start_scatter(b, m, k):
        row = chunk_row(b)
        pltpu.async_copy(rows_bufs[k], numer_sh.at[cdst.at[row]], ssems[k],
                         add=True)
        pltpu.async_copy(ex_bufs[m], denom_sh.at[cdst.at[row]], ssems[k],
                         add=True)

    def wait_scatter(b, m, k):
        row = chunk_row(b)
        pltpu.make_async_copy(rows_bufs[k], numer_sh.at[cdst.at[row]],
                              ssems[k]).wait()
        pltpu.make_async_copy(ex_bufs[m], denom_sh.at[cdst.at[row]],
                              ssems[k]).wait()

    def scale_rows(m, k):
        ex_v = ex_bufs[m]
        rows_v = rows_bufs[k]

        def scale_group(g, _):
            sv = ex_v[pl.ds(g * 16, 16)]
            for l in range(16):
                s = sv[l]
                r = g * 16 + l
                for c in range(H // 16):
                    sl = pl.ds(c * 16, 16)
                    rows_v[r, sl] = rows_v[r, sl] * s
            return 0

        lax.fori_loop(0, MB // 16, scale_group, 0)

    def iteration(b, k, first=False, prefetch=True):
        kn = (k + 1) % 2
        if not first:
            wait_scatter(b - 1, kn, kn)

        @pl.when(jnp.logical_and(b % CH == 0, b // CH + 1 < NCHUNK))
        def _():
            issue_chunk(b // CH + 1)

        wait_gather(b, k)
        if prefetch:
            @pl.when((b + 1) % CH == 0)
            def _():
                wait_chunk((b + 1) // CH)

            start_gather(b + 1, kn)
        scale_rows(k, k)
        start_scatter(b, k, k)
        if prefetch:
            compute_ex(b + 1, kn)

    pltpu.sync_copy(src_hbm.at[wid, 0], csrc.at[pl.ds(0, CH), :])
    pltpu.sync_copy(dst_hbm.at[wid, 0], cdst.at[pl.ds(0, CH), :])
    start_gather(0, 0)
    compute_ex(0, 0)

    iteration(0, 0, first=True)
    iteration(1, 1)

    def body(t, _):
        b = t * 2
        iteration(b, 0)
        iteration(b + 1, 1)
        return 0

    lax.fori_loop(1, NB // 2 - 1, body, 0)

    iteration(NB - 2, 0)
    iteration(NB - 1, 1, prefetch=False)

    wait_scatter(NB - 1, (NB - 1) % 2, (NB - 1) % 2)

    plsc.subcore_barrier()

    pltpu.sync_copy(numer_sh.at[pl.ds(sid * ROWS_PER_TILE, ROWS_PER_TILE), :],
                    numer_out.at[cid, pl.ds(sid * ROWS_PER_TILE, ROWS_PER_TILE), :])

    @pl.when(sid == 0)
    def _():
        pltpu.sync_copy(denom_sh, denom_out.at[cid])


def _edge_pass(src, dst, a_src, a_dst, h_src):
    mesh = plsc.VectorSubcoreMesh(core_axis_name="c", subcore_axis_name="s")
    return pl.kernel(
        _edge_body,
        out_type=[
            jax.ShapeDtypeStruct((NC, N_PAD, H), jnp.float32),
            jax.ShapeDtypeStruct((NC, N_PAD), jnp.float32),
        ],
        mesh=mesh,
        compiler_params=pltpu.CompilerParams(needs_layout_passes=False),
        scratch_types=[
            pltpu.VMEM((N_PAD,), jnp.float32),
            pltpu.VMEM((N_PAD,), jnp.float32),
            pltpu.VMEM((16, MB), jnp.int32),
            pltpu.VMEM((16, MB), jnp.int32),
            pltpu.VMEM((MB,), jnp.float32),
            pltpu.VMEM((MB,), jnp.float32),
            pltpu.VMEM((MB, H), jnp.float32),
            pltpu.VMEM((MB, H), jnp.float32),
            pltpu.VMEM_SHARED((N_PAD, H), jnp.float32),
            pltpu.VMEM_SHARED((N_PAD,), jnp.float32),
            pltpu.SemaphoreType.DMA,
            pltpu.SemaphoreType.DMA,
            pltpu.SemaphoreType.DMA,
            pltpu.SemaphoreType.DMA,
            pltpu.SemaphoreType.DMA,
        ],
    )(src, dst, a_src, a_dst, h_src)


def _mlp_body(num_ref, den_ref, bc_ref, w1_ref, b1_ref, w2_ref, b2_ref, out_ref):
    n = num_ref[0] + num_ref[1]
    d = den_ref[0, 0, 0] + den_ref[1, 0, 0]
    h = n / (d[:, None] + 1e-16) + bc_ref[...]
    h = jnp.tanh(h)
    h = jnp.dot(h, w1_ref[...], preferred_element_type=jnp.float32) + b1_ref[...]
    h = jnp.tanh(h)
    out_ref[...] = (jnp.dot(h, w2_ref[...], preferred_element_type=jnp.float32)
                    + b2_ref[...])


def _mlp(numer, denom, b_conv, W1, b1, W2, b2):
    nb = 25
    bs = N // nb
    return pl.pallas_call(
        _mlp_body,
        grid=(nb,),
        in_specs=[
            pl.BlockSpec((NC, bs, H), lambda i: (0, i, 0)),
            pl.BlockSpec((NC, 1, 1, bs), lambda i: (0, i, 0, 0)),
            pl.BlockSpec((1, H), lambda i: (0, 0)),
            pl.BlockSpec((H, H), lambda i: (0, 0)),
            pl.BlockSpec((1, H), lambda i: (0, 0)),
            pl.BlockSpec((H, O), lambda i: (0, 0)),
            pl.BlockSpec((1, O), lambda i: (0, 0)),
        ],
        out_specs=pl.BlockSpec((bs, O), lambda i: (i, 0)),
        out_shape=jax.ShapeDtypeStruct((N, O), jnp.float32),
    )(numer, denom[:, :N].reshape(NC, nb, 1, bs), b_conv.reshape(1, H), W1,
      b1.reshape(1, H), W2, b2.reshape(1, O))


@jax.jit
def kernel(x, edge_index, W_src, W_dst, att_src, att_dst, b_conv, W1, b1, W2, b2):
    h_src, a_src, a_dst = _projections(x, W_src, W_dst, att_src, att_dst)

    ppw = EPW - N
    src = jnp.concatenate(
        [edge_index[0].astype(jnp.int32).reshape(NW, N),
         jnp.zeros((NW, ppw), jnp.int32)], axis=1).reshape(NW, NCHUNK, CH, MB)
    pad_dst = N + (jnp.arange(ppw, dtype=jnp.int32) % (N_PAD - N))
    dst = jnp.concatenate(
        [edge_index[1].astype(jnp.int32).reshape(NW, N),
         jnp.broadcast_to(pad_dst, (NW, ppw))], axis=1).reshape(NW, NCHUNK, CH, MB)
    a_src_p = jnp.concatenate([a_src, jnp.zeros((N_PAD - N,), jnp.float32)])
    a_dst_p = jnp.concatenate([a_dst, jnp.zeros((N_PAD - N,), jnp.float32)])

    numer, denom = _edge_pass(src, dst, a_src_p, a_dst_p, h_src)

    return _mlp(numer, denom, b_conv, W1, b1, W2, b2)


# speedup vs baseline: 1.5342x; 1.5342x over previous
import jax
import jax.numpy as jnp
from jax import lax
from jax.experimental import pallas as pl
from jax.experimental.pallas import tpu as pltpu
from jax.experimental.pallas import tpu_sc as plsc

N = 10000
E = 320000
D = 128
H = 128
O = 128

NC = 2
NS = 16
NW = NC * NS
MB = 96
NB = 106
EPW = NB * MB
E_PAD = NW * EPW
N_PAD = 10112
ROWS_PER_TILE = N_PAD // NS


def _proj_body(x_ref, ws_ref, wd_ref, as_ref, ad_ref, h_ref, asr_ref, adr_ref):
    x = x_ref[...]
    h = jnp.dot(x, ws_ref[...], preferred_element_type=jnp.float32)
    h_ref[...] = h
    a_s = jnp.dot(h, as_ref[...].T, preferred_element_type=jnp.float32)
    wd_v = jnp.dot(wd_ref[...], ad_ref[...].T, preferred_element_type=jnp.float32)
    a_d = jnp.dot(x, wd_v, preferred_element_type=jnp.float32)
    asr_ref[...] = a_s.reshape(asr_ref.shape)
    adr_ref[...] = a_d.reshape(adr_ref.shape)


def _projections(x, W_src, W_dst, att_src, att_dst):
    nb = 25
    bs = N // nb
    h, a_s, a_d = pl.pallas_call(
        _proj_body,
        grid=(nb,),
        in_specs=[
            pl.BlockSpec((bs, D), lambda i: (i, 0)),
            pl.BlockSpec((D, H), lambda i: (0, 0)),
            pl.BlockSpec((D, H), lambda i: (0, 0)),
            pl.BlockSpec((1, H), lambda i: (0, 0)),
            pl.BlockSpec((1, H), lambda i: (0, 0)),
        ],
        out_specs=[
            pl.BlockSpec((bs, H), lambda i: (i, 0)),
            pl.BlockSpec((1, 1, bs), lambda i: (i, 0, 0)),
            pl.BlockSpec((1, 1, bs), lambda i: (i, 0, 0)),
        ],
        out_shape=[
            jax.ShapeDtypeStruct((N, H), jnp.float32),
            jax.ShapeDtypeStruct((nb, 1, bs), jnp.float32),
            jax.ShapeDtypeStruct((nb, 1, bs), jnp.float32),
        ],
    )(x, W_src, W_dst, att_src.reshape(1, H), att_dst.reshape(1, H))
    return h, a_s.reshape(N), a_d.reshape(N)


def _edge_body(src_hbm, dst_hbm, asrc_hbm, adst_hbm, h_hbm,
               numer_out, denom_out,
               asrc_v, adst_v,
               si0, si1, di0, di1,
               ex0, ex1, rows0, rows1,
               numer_sh, denom_sh,
               g0, g1, s0, s1, is0, is1, id0, id1):
    cid = lax.axis_index("c")
    sid = lax.axis_index("s")
    wid = sid * NC + cid
    src_bufs = [si0, si1]
    dst_bufs = [di0, di1]
    ex_bufs = [ex0, ex1]
    rows_bufs = [rows0, rows1]
    gsems = [g0, g1]
    ssems = [s0, s1]
    isems_s = [is0, is1]
    isems_d = [id0, id1]
    ebase = wid * EPW

    zv = jnp.zeros((16,), jnp.float32)

    def zero_rows(r, _):
        for c in range(H // 16):
            rows0[r, pl.ds(c * 16, 16)] = zv
        return 0

    lax.fori_loop(0, MB, zero_rows, 0)
    for g in range(MB // 16):
        ex0[pl.ds(g * 16, 16)] = zv
    tbase = sid * ROWS_PER_TILE
    for j in range(ROWS_PER_TILE // MB):
        pltpu.sync_copy(rows0, numer_sh.at[pl.ds(tbase + j * MB, MB), :])
        pltpu.sync_copy(ex0, denom_sh.at[pl.ds(tbase + j * MB, MB)])
    rem = ROWS_PER_TILE % MB
    rbase = tbase + (ROWS_PER_TILE // MB) * MB
    pltpu.sync_copy(rows0.at[pl.ds(0, rem), :],
                    numer_sh.at[pl.ds(rbase, rem), :])
    pltpu.sync_copy(ex0.at[pl.ds(0, rem)], denom_sh.at[pl.ds(rbase, rem)])

    pltpu.sync_copy(asrc_hbm, asrc_v)
    pltpu.sync_copy(adst_hbm, adst_v)

    plsc.subcore_barrier()

    def start_src_idx(b, m):
        pltpu.async_copy(src_hbm.at[pl.ds(ebase + b * MB, MB)], src_bufs[m],
                         isems_s[m])

    def wait_src_idx(b, m):
        pltpu.make_async_copy(src_hbm.at[pl.ds(ebase + b * MB, MB)],
                              src_bufs[m], isems_s[m]).wait()

    def start_dst_idx(b, m):
        pltpu.async_copy(dst_hbm.at[pl.ds(ebase + b * MB, MB)], dst_bufs[m],
                         isems_d[m])

    def wait_dst_idx(b, m):
        pltpu.make_async_copy(dst_hbm.at[pl.ds(ebase + b * MB, MB)],
                              dst_bufs[m], isems_d[m]).wait()

    def compute_ex(m):
        sv, dv, ev = src_bufs[m], dst_bufs[m], ex_bufs[m]
        for g in range(MB // 16):
            si = sv[pl.ds(g * 16, 16)]
            di = dv[pl.ds(g * 16, 16)]
            av = plsc.load_gather(asrc_v, [si])
            bv = plsc.load_gather(adst_v, [di])
            al = av + bv
            al = jnp.where(al >= 0.0, al, 0.2 * al)
            ev[pl.ds(g * 16, 16)] = jnp.exp(al)

    def start_gather(m, k):
        pltpu.async_copy(h_hbm.at[src_bufs[m]], rows_bufs[k], gsems[k])

    def wait_gather(m, k):
        pltpu.make_async_copy(h_hbm.at[src_bufs[m]], rows_bufs[k],
                              gsems[k]).wait()

    def start_scatter(m, k):
        pltpu.async_copy(rows_bufs[k], numer_sh.at[dst_bufs[m]], ssems[k],
                         add=True)
        pltpu.async_copy(ex_bufs[m], denom_sh.at[dst_bufs[m]], ssems[k],
                         add=True)

    def wait_scatter(m, k):
        pltpu.make_async_copy(rows_bufs[k], numer_sh.at[dst_bufs[m]],
                              ssems[k]).wait()
        pltpu.make_async_copy(ex_bufs[m], denom_sh.at[dst_bufs[m]],
                              ssems[k]).wait()

    def scale_rows(m, k):
        ex_v = ex_bufs[m]
        rows_v = rows_bufs[k]

        def scale_group(g, _):
            sv = ex_v[pl.ds(g * 16, 16)]
            for l in range(16):
                s = sv[l]
                r = g * 16 + l
                for c in range(H // 16):
                    sl = pl.ds(c * 16, 16)
                    rows_v[r, sl] = rows_v[r, sl] * s
            return 0

        lax.fori_loop(0, MB // 16, scale_group, 0)

    def iteration(b, k, first=False, prefetch=True):
        kn = (k + 1) % 2
        if prefetch:
            start_src_idx(b + 1, kn)
        if not first:
            wait_scatter(kn, kn)
        if prefetch:
            start_dst_idx(b + 1, kn)
        wait_gather(k, k)
        if prefetch:
            wait_src_idx(b + 1, kn)
            start_gather(kn, kn)
        scale_rows(k, k)
        start_scatter(k, k)
        if prefetch:
            wait_dst_idx(b + 1, kn)
            compute_ex(kn)

    pltpu.sync_copy(src_hbm.at[pl.ds(ebase, MB)], src_bufs[0])
    pltpu.sync_copy(dst_hbm.at[pl.ds(ebase, MB)], dst_bufs[0])
    start_gather(0, 0)
    compute_ex(0)

    iteration(0, 0, first=True)
    iteration(1, 1)

    def body(t, _):
        b = t * 2
        iteration(b, 0)
        iteration(b + 1, 1)
        return 0

    lax.fori_loop(1, NB // 2 - 1, body, 0)

    iteration(NB - 2, 0)
    iteration(NB - 1, 1, prefetch=False)

    wait_scatter((NB - 1) % 2, (NB - 1) % 2)

    plsc.subcore_barrier()

    pltpu.sync_copy(numer_sh.at[pl.ds(sid * ROWS_PER_TILE, ROWS_PER_TILE), :],
                    numer_out.at[cid, pl.ds(sid * ROWS_PER_TILE, ROWS_PER_TILE), :])

    @pl.when(sid == 0)
    def _():
        pltpu.sync_copy(denom_sh, denom_out.at[cid])


def _edge_pass(src, dst, a_src, a_dst, h_src):
    mesh = plsc.VectorSubcoreMesh(core_axis_name="c", subcore_axis_name="s")
    return pl.kernel(
        _edge_body,
        out_type=[
            jax.ShapeDtypeStruct((NC, N_PAD, H), jnp.float32),
            jax.ShapeDtypeStruct((NC, N_PAD), jnp.float32),
        ],
        mesh=mesh,
        compiler_params=pltpu.CompilerParams(needs_layout_passes=False),
        scratch_types=[
            pltpu.VMEM((N_PAD,), jnp.float32),
            pltpu.VMEM((N_PAD,), jnp.float32),
            pltpu.VMEM((MB,), jnp.int32),
            pltpu.VMEM((MB,), jnp.int32),
            pltpu.VMEM((MB,), jnp.int32),
            pltpu.VMEM((MB,), jnp.int32),
            pltpu.VMEM((MB,), jnp.float32),
            pltpu.VMEM((MB,), jnp.float32),
            pltpu.VMEM((MB, H), jnp.float32),
            pltpu.VMEM((MB, H), jnp.float32),
            pltpu.VMEM_SHARED((N_PAD, H), jnp.float32),
            pltpu.VMEM_SHARED((N_PAD,), jnp.float32),
            pltpu.SemaphoreType.DMA,
            pltpu.SemaphoreType.DMA,
            pltpu.SemaphoreType.DMA,
            pltpu.SemaphoreType.DMA,
            pltpu.SemaphoreType.DMA,
            pltpu.SemaphoreType.DMA,
            pltpu.SemaphoreType.DMA,
            pltpu.SemaphoreType.DMA,
        ],
    )(src, dst, a_src, a_dst, h_src)


def _mlp_body(num_ref, den_ref, bc_ref, w1_ref, b1_ref, w2_ref, b2_ref, out_ref):
    n = num_ref[0] + num_ref[1]
    d = den_ref[0, 0, 0] + den_ref[1, 0, 0]
    h = n / (d[:, None] + 1e-16) + bc_ref[...]
    h = jnp.tanh(h)
    h = jnp.dot(h, w1_ref[...], preferred_element_type=jnp.float32) + b1_ref[...]
    h = jnp.tanh(h)
    out_ref[...] = (jnp.dot(h, w2_ref[...], preferred_element_type=jnp.float32)
                    + b2_ref[...])


def _mlp(numer, denom, b_conv, W1, b1, W2, b2):
    nb = 25
    bs = N // nb
    return pl.pallas_call(
        _mlp_body,
        grid=(nb,),
        in_specs=[
            pl.BlockSpec((NC, bs, H), lambda i: (0, i, 0)),
            pl.BlockSpec((NC, 1, 1, bs), lambda i: (0, i, 0, 0)),
            pl.BlockSpec((1, H), lambda i: (0, 0)),
            pl.BlockSpec((H, H), lambda i: (0, 0)),
            pl.BlockSpec((1, H), lambda i: (0, 0)),
            pl.BlockSpec((H, O), lambda i: (0, 0)),
            pl.BlockSpec((1, O), lambda i: (0, 0)),
        ],
        out_specs=pl.BlockSpec((bs, O), lambda i: (i, 0)),
        out_shape=jax.ShapeDtypeStruct((N, O), jnp.float32),
    )(numer, denom[:, :N].reshape(NC, nb, 1, bs), b_conv.reshape(1, H), W1,
      b1.reshape(1, H), W2, b2.reshape(1, O))


@jax.jit
def kernel(x, edge_index, W_src, W_dst, att_src, att_dst, b_conv, W1, b1, W2, b2):
    h_src, a_src, a_dst = _projections(x, W_src, W_dst, att_src, att_dst)

    ppw = EPW - N
    src = jnp.concatenate(
        [edge_index[0].astype(jnp.int32).reshape(NW, N),
         jnp.zeros((NW, ppw), jnp.int32)], axis=1).reshape(-1)
    pad_dst = N + (jnp.arange(ppw, dtype=jnp.int32) % (N_PAD - N))
    dst = jnp.concatenate(
        [edge_index[1].astype(jnp.int32).reshape(NW, N),
         jnp.broadcast_to(pad_dst, (NW, ppw))], axis=1).reshape(-1)
    a_src_p = jnp.concatenate([a_src, jnp.zeros((N_PAD - N,), jnp.float32)])
    a_dst_p = jnp.concatenate([a_dst, jnp.zeros((N_PAD - N,), jnp.float32)])

    numer, denom = _edge_pass(src, dst, a_src_p, a_dst_p, h_src)

    return _mlp(numer, denom, b_conv, W1, b1, W2, b2)


# speedup vs baseline: 1.5370x; 1.0018x over previous
import jax
import jax.numpy as jnp
from jax import lax
from jax.experimental import pallas as pl
from jax.experimental.pallas import tpu as pltpu
from jax.experimental.pallas import tpu_sc as plsc

N = 10000
E = 320000
D = 128
H = 128
O = 128

NC = 2
NS = 16
NW = NC * NS
MB = 64
NB = 159
EPW = NB * MB
E_PAD = NW * EPW
N_PAD = 10112
ROWS_PER_TILE = N_PAD // NS


def _proj_body(x_ref, ws_ref, wd_ref, as_ref, ad_ref, h_ref, asr_ref, adr_ref):
    x = x_ref[...]
    h = jnp.dot(x, ws_ref[...], preferred_element_type=jnp.float32)
    h_ref[...] = h
    a_s = jnp.dot(h, as_ref[...].T, preferred_element_type=jnp.float32)
    wd_v = jnp.dot(wd_ref[...], ad_ref[...].T, preferred_element_type=jnp.float32)
    a_d = jnp.dot(x, wd_v, preferred_element_type=jnp.float32)
    asr_ref[...] = a_s.reshape(asr_ref.shape)
    adr_ref[...] = a_d.reshape(adr_ref.shape)


def _projections(x, W_src, W_dst, att_src, att_dst):
    nb = 25
    bs = N // nb
    h, a_s, a_d = pl.pallas_call(
        _proj_body,
        grid=(nb,),
        in_specs=[
            pl.BlockSpec((bs, D), lambda i: (i, 0)),
            pl.BlockSpec((D, H), lambda i: (0, 0)),
            pl.BlockSpec((D, H), lambda i: (0, 0)),
            pl.BlockSpec((1, H), lambda i: (0, 0)),
            pl.BlockSpec((1, H), lambda i: (0, 0)),
        ],
        out_specs=[
            pl.BlockSpec((bs, H), lambda i: (i, 0)),
            pl.BlockSpec((1, 1, bs), lambda i: (i, 0, 0)),
            pl.BlockSpec((1, 1, bs), lambda i: (i, 0, 0)),
        ],
        out_shape=[
            jax.ShapeDtypeStruct((N, H), jnp.float32),
            jax.ShapeDtypeStruct((nb, 1, bs), jnp.float32),
            jax.ShapeDtypeStruct((nb, 1, bs), jnp.float32),
        ],
    )(x, W_src, W_dst, att_src.reshape(1, H), att_dst.reshape(1, H))
    return h, a_s.reshape(N), a_d.reshape(N)


def _edge_body(src_hbm, dst_hbm, asrc_hbm, adst_hbm, h_hbm,
               numer_out, denom_out,
               asrc_v, adst_v,
               si0, si1, si2, di0, di1, di2,
               ex0, ex1, ex2, rows0, rows1, rows2,
               numer_sh, denom_sh,
               g0, g1, g2, s0, s1, s2,
               is0, is1, is2, id0, id1, id2):
    cid = lax.axis_index("c")
    sid = lax.axis_index("s")
    wid = sid * NC + cid
    src_bufs = [si0, si1, si2]
    dst_bufs = [di0, di1, di2]
    ex_bufs = [ex0, ex1, ex2]
    rows_bufs = [rows0, rows1, rows2]
    gsems = [g0, g1, g2]
    ssems = [s0, s1, s2]
    isems_s = [is0, is1, is2]
    isems_d = [id0, id1, id2]
    ebase = wid * EPW

    zv = jnp.zeros((16,), jnp.float32)

    def zero_rows(r, _):
        for c in range(H // 16):
            rows0[r, pl.ds(c * 16, 16)] = zv
        return 0

    lax.fori_loop(0, MB, zero_rows, 0)
    for g in range(MB // 16):
        ex0[pl.ds(g * 16, 16)] = zv
    tbase = sid * ROWS_PER_TILE
    for j in range(ROWS_PER_TILE // MB):
        pltpu.sync_copy(rows0, numer_sh.at[pl.ds(tbase + j * MB, MB), :])
        pltpu.sync_copy(ex0, denom_sh.at[pl.ds(tbase + j * MB, MB)])
    rem = ROWS_PER_TILE % MB
    rbase = tbase + (ROWS_PER_TILE // MB) * MB
    pltpu.sync_copy(rows0.at[pl.ds(0, rem), :],
                    numer_sh.at[pl.ds(rbase, rem), :])
    pltpu.sync_copy(ex0.at[pl.ds(0, rem)], denom_sh.at[pl.ds(rbase, rem)])

    pltpu.sync_copy(asrc_hbm, asrc_v)
    pltpu.sync_copy(adst_hbm, adst_v)

    plsc.subcore_barrier()

    def start_src_idx(b, m):
        pltpu.async_copy(src_hbm.at[pl.ds(ebase + b * MB, MB)], src_bufs[m],
                         isems_s[m])

    def wait_src_idx(b, m):
        pltpu.make_async_copy(src_hbm.at[pl.ds(ebase + b * MB, MB)],
                              src_bufs[m], isems_s[m]).wait()

    def start_dst_idx(b, m):
        pltpu.async_copy(dst_hbm.at[pl.ds(ebase + b * MB, MB)], dst_bufs[m],
                         isems_d[m])

    def wait_dst_idx(b, m):
        pltpu.make_async_copy(dst_hbm.at[pl.ds(ebase + b * MB, MB)],
                              dst_bufs[m], isems_d[m]).wait()

    def compute_ex(m):
        sv, dv, ev = src_bufs[m], dst_bufs[m], ex_bufs[m]
        for g in range(MB // 16):
            si = sv[pl.ds(g * 16, 16)]
            di = dv[pl.ds(g * 16, 16)]
            av = plsc.load_gather(asrc_v, [si])
            bv = plsc.load_gather(adst_v, [di])
            al = av + bv
            al = jnp.where(al >= 0.0, al, 0.2 * al)
            ev[pl.ds(g * 16, 16)] = jnp.exp(al)

    def start_gather(m, k):
        pltpu.async_copy(h_hbm.at[src_bufs[m]], rows_bufs[k], gsems[k])

    def wait_gather(m, k):
        pltpu.make_async_copy(h_hbm.at[src_bufs[m]], rows_bufs[k],
                              gsems[k]).wait()

    def start_scatter(m, k):
        pltpu.async_copy(rows_bufs[k], numer_sh.at[dst_bufs[m]], ssems[k],
                         add=True)
        pltpu.async_copy(ex_bufs[m], denom_sh.at[dst_bufs[m]], ssems[k],
                         add=True)

    def wait_scatter(m, k):
        pltpu.make_async_copy(rows_bufs[k], numer_sh.at[dst_bufs[m]],
                              ssems[k]).wait()
        pltpu.make_async_copy(ex_bufs[m], denom_sh.at[dst_bufs[m]],
                              ssems[k]).wait()

    def scale_rows(m, k):
        ex_v = ex_bufs[m]
        rows_v = rows_bufs[k]

        def scale_group(g, _):
            sv = ex_v[pl.ds(g * 16, 16)]
            for l in range(16):
                s = sv[l]
                r = g * 16 + l
                for c in range(H // 16):
                    sl = pl.ds(c * 16, 16)
                    rows_v[r, sl] = rows_v[r, sl] * s
            return 0

        lax.fori_loop(0, MB // 16, scale_group, 0)

    def iteration(b, k, first=False, start_next=False,
                  idx2=True, ex1=True, g2=True):
        kn = (k + 1) % 3
        kp = (k + 2) % 3
        if not first:
            wait_scatter(kp, kp)
        if idx2:
            start_src_idx(b + 2, kp)
            start_dst_idx(b + 2, kp)
        wait_gather(k, k)
        scale_rows(k, k)
        start_scatter(k, k)
        if ex1:
            wait_dst_idx(b + 1, kn)
            compute_ex(kn)
        if start_next:
            start_gather(kn, kn)
        if g2:
            wait_src_idx(b + 2, kp)
            start_gather(kp, kp)

    pltpu.sync_copy(src_hbm.at[pl.ds(ebase, MB)], src_bufs[0])
    pltpu.sync_copy(dst_hbm.at[pl.ds(ebase, MB)], dst_bufs[0])
    start_src_idx(1, 1)
    start_dst_idx(1, 1)
    start_gather(0, 0)
    compute_ex(0)

    wait_src_idx(1, 1)
    iteration(0, 0, first=True, start_next=True)
    iteration(1, 1)
    iteration(2, 2)

    def body(t, _):
        b = t * 3
        iteration(b, 0)
        iteration(b + 1, 1)
        iteration(b + 2, 2)
        return 0

    lax.fori_loop(1, NB // 3 - 1, body, 0)

    iteration(NB - 3, 0)
    iteration(NB - 2, 1, idx2=False, g2=False)
    iteration(NB - 1, 2, idx2=False, ex1=False, g2=False)

    wait_scatter((NB - 1) % 3, (NB - 1) % 3)

    plsc.subcore_barrier()

    pltpu.sync_copy(numer_sh.at[pl.ds(sid * ROWS_PER_TILE, ROWS_PER_TILE), :],
                    numer_out.at[cid, pl.ds(sid * ROWS_PER_TILE, ROWS_PER_TILE), :])

    @pl.when(sid == 0)
    def _():
        pltpu.sync_copy(denom_sh, denom_out.at[cid])


def _edge_pass(src, dst, a_src, a_dst, h_src):
    mesh = plsc.VectorSubcoreMesh(core_axis_name="c", subcore_axis_name="s")
    return pl.kernel(
        _edge_body,
        out_type=[
            jax.ShapeDtypeStruct((NC, N_PAD, H), jnp.float32),
            jax.ShapeDtypeStruct((NC, N_PAD), jnp.float32),
        ],
        mesh=mesh,
        compiler_params=pltpu.CompilerParams(needs_layout_passes=False),
        scratch_types=[
            pltpu.VMEM((N_PAD,), jnp.float32),
            pltpu.VMEM((N_PAD,), jnp.float32),
            pltpu.VMEM((MB,), jnp.int32),
            pltpu.VMEM((MB,), jnp.int32),
            pltpu.VMEM((MB,), jnp.int32),
            pltpu.VMEM((MB,), jnp.int32),
            pltpu.VMEM((MB,), jnp.int32),
            pltpu.VMEM((MB,), jnp.int32),
            pltpu.VMEM((MB,), jnp.float32),
            pltpu.VMEM((MB,), jnp.float32),
            pltpu.VMEM((MB,), jnp.float32),
            pltpu.VMEM((MB, H), jnp.float32),
            pltpu.VMEM((MB, H), jnp.float32),
            pltpu.VMEM((MB, H), jnp.float32),
            pltpu.VMEM_SHARED((N_PAD, H), jnp.float32),
            pltpu.VMEM_SHARED((N_PAD,), jnp.float32),
            pltpu.SemaphoreType.DMA,
            pltpu.SemaphoreType.DMA,
            pltpu.SemaphoreType.DMA,
            pltpu.SemaphoreType.DMA,
            pltpu.SemaphoreType.DMA,
            pltpu.SemaphoreType.DMA,
            pltpu.SemaphoreType.DMA,
            pltpu.SemaphoreType.DMA,
            pltpu.SemaphoreType.DMA,
            pltpu.SemaphoreType.DMA,
            pltpu.SemaphoreType.DMA,
            pltpu.SemaphoreType.DMA,
        ],
    )(src, dst, a_src, a_dst, h_src)


def _mlp_body(num_ref, den_ref, bc_ref, w1_ref, b1_ref, w2_ref, b2_ref, out_ref):
    n = num_ref[0] + num_ref[1]
    d = den_ref[0, 0, 0] + den_ref[1, 0, 0]
    h = n / (d[:, None] + 1e-16) + bc_ref[...]
    h = jnp.tanh(h)
    h = jnp.dot(h, w1_ref[...], preferred_element_type=jnp.float32) + b1_ref[...]
    h = jnp.tanh(h)
    out_ref[...] = (jnp.dot(h, w2_ref[...], preferred_element_type=jnp.float32)
                    + b2_ref[...])


def _mlp(numer, denom, b_conv, W1, b1, W2, b2):
    nb = 25
    bs = N // nb
    return pl.pallas_call(
        _mlp_body,
        grid=(nb,),
        in_specs=[
            pl.BlockSpec((NC, bs, H), lambda i: (0, i, 0)),
            pl.BlockSpec((NC, 1, 1, bs), lambda i: (0, i, 0, 0)),
            pl.BlockSpec((1, H), lambda i: (0, 0)),
            pl.BlockSpec((H, H), lambda i: (0, 0)),
            pl.BlockSpec((1, H), lambda i: (0, 0)),
            pl.BlockSpec((H, O), lambda i: (0, 0)),
            pl.BlockSpec((1, O), lambda i: (0, 0)),
        ],
        out_specs=pl.BlockSpec((bs, O), lambda i: (i, 0)),
        out_shape=jax.ShapeDtypeStruct((N, O), jnp.float32),
    )(numer, denom[:, :N].reshape(NC, nb, 1, bs), b_conv.reshape(1, H), W1,
      b1.reshape(1, H), W2, b2.reshape(1, O))


@jax.jit
def kernel(x, edge_index, W_src, W_dst, att_src, att_dst, b_conv, W1, b1, W2, b2):
    h_src, a_src, a_dst = _projections(x, W_src, W_dst, att_src, att_dst)

    ppw = EPW - N
    src = jnp.concatenate(
        [edge_index[0].astype(jnp.int32).reshape(NW, N),
         jnp.zeros((NW, ppw), jnp.int32)], axis=1).reshape(-1)
    pad_dst = N + (jnp.arange(ppw, dtype=jnp.int32) % (N_PAD - N))
    dst = jnp.concatenate(
        [edge_index[1].astype(jnp.int32).reshape(NW, N),
         jnp.broadcast_to(pad_dst, (NW, ppw))], axis=1).reshape(-1)
    a_src_p = jnp.concatenate([a_src, jnp.zeros((N_PAD - N,), jnp.float32)])
    a_dst_p = jnp.concatenate([a_dst, jnp.zeros((N_PAD - N,), jnp.float32)])

    numer, denom = _edge_pass(src, dst, a_src_p, a_dst_p, h_src)

    return _mlp(numer, denom, b_conv, W1, b1, W2, b2)


# speedup vs baseline: 2.8203x; 1.8350x over previous
import jax
import jax.numpy as jnp
from jax import lax
from jax.experimental import pallas as pl
from jax.experimental.pallas import tpu as pltpu
from jax.experimental.pallas import tpu_sc as plsc

N = 10000
E = 320000
D = 128
H = 128
O = 128

NC = 2
NS = 16
NW = NC * NS
MB = 80
NB = 125
EPW = NB * MB
E_PAD = NW * EPW
N_PAD = 10112
ROWS_PER_TILE = N_PAD // NS


def _proj_body(x_ref, ws_ref, wd_ref, as_ref, ad_ref, h_ref, asr_ref, adr_ref):
    x = x_ref[...]
    h = jnp.dot(x, ws_ref[...], preferred_element_type=jnp.float32)
    h_ref[...] = h
    a_s = jnp.dot(h, as_ref[...].T, preferred_element_type=jnp.float32)
    wd_v = jnp.dot(wd_ref[...], ad_ref[...].T, preferred_element_type=jnp.float32)
    a_d = jnp.dot(x, wd_v, preferred_element_type=jnp.float32)
    asr_ref[...] = a_s.reshape(asr_ref.shape)
    adr_ref[...] = a_d.reshape(adr_ref.shape)


def _projections(x, W_src, W_dst, att_src, att_dst):
    nb = 25
    bs = N // nb
    h, a_s, a_d = pl.pallas_call(
        _proj_body,
        grid=(nb,),
        in_specs=[
            pl.BlockSpec((bs, D), lambda i: (i, 0)),
            pl.BlockSpec((D, H), lambda i: (0, 0)),
            pl.BlockSpec((D, H), lambda i: (0, 0)),
            pl.BlockSpec((1, H), lambda i: (0, 0)),
            pl.BlockSpec((1, H), lambda i: (0, 0)),
        ],
        out_specs=[
            pl.BlockSpec((bs, H), lambda i: (i, 0)),
            pl.BlockSpec((1, 1, bs), lambda i: (i, 0, 0)),
            pl.BlockSpec((1, 1, bs), lambda i: (i, 0, 0)),
        ],
        out_shape=[
            jax.ShapeDtypeStruct((N, H), jnp.float32),
            jax.ShapeDtypeStruct((nb, 1, bs), jnp.float32),
            jax.ShapeDtypeStruct((nb, 1, bs), jnp.float32),
        ],
    )(x, W_src, W_dst, att_src.reshape(1, H), att_dst.reshape(1, H))
    return h, a_s.reshape(N), a_d.reshape(N)


def _edge_body(src_hbm, dst_hbm, asrc_hbm, adst_hbm, h_hbm,
               numer_out, denom_out,
               asrc_v, adst_v,
               si0, si1, di0, di1,
               ex0, ex1, rows0, rows1,
               numer_sh, denom_sh,
               g0, g1, s0, s1, is0, is1, id0, id1):
    cid = lax.axis_index("c")
    sid = lax.axis_index("s")
    wid = sid * NC + cid
    src_bufs = [si0, si1]
    dst_bufs = [di0, di1]
    ex_bufs = [ex0, ex1]
    rows_bufs = [rows0, rows1]
    gsems = [g0, g1]
    ssems = [s0, s1]
    isems_s = [is0, is1]
    isems_d = [id0, id1]
    ebase = wid * EPW

    zv = jnp.zeros((16,), jnp.float32)

    def zero_rows(r, _):
        for c in range(H // 16):
            rows0[r, pl.ds(c * 16, 16)] = zv
        return 0

    lax.fori_loop(0, MB, zero_rows, 0)
    for g in range(MB // 16):
        ex0[pl.ds(g * 16, 16)] = zv
    tbase = sid * ROWS_PER_TILE
    for j in range(ROWS_PER_TILE // MB):
        pltpu.sync_copy(rows0, numer_sh.at[pl.ds(tbase + j * MB, MB), :])
        pltpu.sync_copy(ex0, denom_sh.at[pl.ds(tbase + j * MB, MB)])
    rem = ROWS_PER_TILE % MB
    rbase = tbase + (ROWS_PER_TILE // MB) * MB
    pltpu.sync_copy(rows0.at[pl.ds(0, rem), :],
                    numer_sh.at[pl.ds(rbase, rem), :])
    pltpu.sync_copy(ex0.at[pl.ds(0, rem)], denom_sh.at[pl.ds(rbase, rem)])

    pltpu.sync_copy(asrc_hbm, asrc_v)
    pltpu.sync_copy(adst_hbm, adst_v)

    plsc.subcore_barrier()

    def start_src_idx(b, m):
        pltpu.async_copy(src_hbm.at[pl.ds(ebase + b * MB, MB)], src_bufs[m],
                         isems_s[m])

    def wait_src_idx(b, m):
        pltpu.make_async_copy(src_hbm.at[pl.ds(ebase + b * MB, MB)],
                              src_bufs[m], isems_s[m]).wait()

    def start_dst_idx(b, m):
        pltpu.async_copy(dst_hbm.at[pl.ds(ebase + b * MB, MB)], dst_bufs[m],
                         isems_d[m])

    def wait_dst_idx(b, m):
        pltpu.make_async_copy(dst_hbm.at[pl.ds(ebase + b * MB, MB)],
                              dst_bufs[m], isems_d[m]).wait()

    def compute_ex(m):
        sv, dv, ev = src_bufs[m], dst_bufs[m], ex_bufs[m]
        for g in range(MB // 16):
            si = sv[pl.ds(g * 16, 16)]
            di = dv[pl.ds(g * 16, 16)]
            av = plsc.load_gather(asrc_v, [si])
            bv = plsc.load_gather(adst_v, [di])
            al = av + bv
            al = jnp.where(al >= 0.0, al, 0.2 * al)
            ev[pl.ds(g * 16, 16)] = jnp.exp(al)

    def start_gather(m, k):
        pltpu.async_copy(h_hbm.at[src_bufs[m]], rows_bufs[k], gsems[k])

    def wait_gather(m, k):
        pltpu.make_async_copy(h_hbm.at[src_bufs[m]], rows_bufs[k],
                              gsems[k]).wait()

    def start_scatter(m, k):
        pltpu.async_copy(rows_bufs[k], numer_sh.at[dst_bufs[m]], ssems[k],
                         add=True)
        pltpu.async_copy(ex_bufs[m], denom_sh.at[dst_bufs[m]], ssems[k],
                         add=True)

    def wait_scatter(m, k):
        pltpu.make_async_copy(rows_bufs[k], numer_sh.at[dst_bufs[m]],
                              ssems[k]).wait()
        pltpu.make_async_copy(ex_bufs[m], denom_sh.at[dst_bufs[m]],
                              ssems[k]).wait()

    def scale_rows(m, k):
        ex_v = ex_bufs[m]
        rows_v = rows_bufs[k]

        def scale_group(g, _):
            sv = ex_v[pl.ds(g * 16, 16)]
            for l in range(16):
                s = sv[l]
                r = g * 16 + l
                for c in range(H // 16):
                    sl = pl.ds(c * 16, 16)
                    rows_v[r, sl] = rows_v[r, sl] * s
            return 0

        lax.fori_loop(0, MB // 16, scale_group, 0)

    def iteration(b, k, first=False, prefetch=True):
        kn = (k + 1) % 2
        if prefetch:
            start_src_idx(b + 1, kn)
        if not first:
            wait_scatter(kn, kn)
        if prefetch:
            start_dst_idx(b + 1, kn)
        wait_gather(k, k)
        if prefetch:
            wait_src_idx(b + 1, kn)
            start_gather(kn, kn)
        scale_rows(k, k)
        start_scatter(k, k)
        if prefetch:
            wait_dst_idx(b + 1, kn)
            compute_ex(kn)

    pltpu.sync_copy(src_hbm.at[pl.ds(ebase, MB)], src_bufs[0])
    pltpu.sync_copy(dst_hbm.at[pl.ds(ebase, MB)], dst_bufs[0])
    start_gather(0, 0)
    compute_ex(0)

    iteration(0, 0, first=True)
    iteration(1, 1)

    def body(t, _):
        b = t * 2
        iteration(b, 0)
        iteration(b + 1, 1)
        return 0

    lax.fori_loop(1, NB // 2, body, 0)

    iteration(NB - 1, (NB - 1) % 2, prefetch=False)

    wait_scatter((NB - 1) % 2, (NB - 1) % 2)

    plsc.subcore_barrier()

    pltpu.sync_copy(numer_sh.at[pl.ds(sid * ROWS_PER_TILE, ROWS_PER_TILE), :],
                    numer_out.at[cid, pl.ds(sid * ROWS_PER_TILE, ROWS_PER_TILE), :])

    @pl.when(sid == 0)
    def _():
        pltpu.sync_copy(denom_sh, denom_out.at[cid])


def _edge_pass(src, dst, a_src, a_dst, h_src):
    mesh = plsc.VectorSubcoreMesh(core_axis_name="c", subcore_axis_name="s")
    return pl.kernel(
        _edge_body,
        out_type=[
            jax.ShapeDtypeStruct((NC, N_PAD, H), jnp.float32),
            jax.ShapeDtypeStruct((NC, N_PAD), jnp.float32),
        ],
        mesh=mesh,
        compiler_params=pltpu.CompilerParams(needs_layout_passes=False),
        scratch_types=[
            pltpu.VMEM((N_PAD,), jnp.float32),
            pltpu.VMEM((N_PAD,), jnp.float32),
            pltpu.VMEM((MB,), jnp.int32),
            pltpu.VMEM((MB,), jnp.int32),
            pltpu.VMEM((MB,), jnp.int32),
            pltpu.VMEM((MB,), jnp.int32),
            pltpu.VMEM((MB,), jnp.float32),
            pltpu.VMEM((MB,), jnp.float32),
            pltpu.VMEM((MB, H), jnp.float32),
            pltpu.VMEM((MB, H), jnp.float32),
            pltpu.VMEM_SHARED((N_PAD, H), jnp.float32),
            pltpu.VMEM_SHARED((N_PAD,), jnp.float32),
            pltpu.SemaphoreType.DMA,
            pltpu.SemaphoreType.DMA,
            pltpu.SemaphoreType.DMA,
            pltpu.SemaphoreType.DMA,
            pltpu.SemaphoreType.DMA,
            pltpu.SemaphoreType.DMA,
            pltpu.SemaphoreType.DMA,
            pltpu.SemaphoreType.DMA,
        ],
    )(src, dst, a_src, a_dst, h_src)


def _mlp_body(num_ref, den_ref, bc_ref, w1_ref, b1_ref, w2_ref, b2_ref, out_ref):
    n = num_ref[0] + num_ref[1]
    d = den_ref[0, 0, 0] + den_ref[1, 0, 0]
    h = n / (d[:, None] + 1e-16) + bc_ref[...]
    h = jnp.tanh(h)
    h = jnp.dot(h, w1_ref[...], preferred_element_type=jnp.float32) + b1_ref[...]
    h = jnp.tanh(h)
    out_ref[...] = (jnp.dot(h, w2_ref[...], preferred_element_type=jnp.float32)
                    + b2_ref[...])


def _mlp(numer, denom, b_conv, W1, b1, W2, b2):
    nb = 25
    bs = N // nb
    return pl.pallas_call(
        _mlp_body,
        grid=(nb,),
        in_specs=[
            pl.BlockSpec((NC, bs, H), lambda i: (0, i, 0)),
            pl.BlockSpec((NC, 1, 1, bs), lambda i: (0, i, 0, 0)),
            pl.BlockSpec((1, H), lambda i: (0, 0)),
            pl.BlockSpec((H, H), lambda i: (0, 0)),
            pl.BlockSpec((1, H), lambda i: (0, 0)),
            pl.BlockSpec((H, O), lambda i: (0, 0)),
            pl.BlockSpec((1, O), lambda i: (0, 0)),
        ],
        out_specs=pl.BlockSpec((bs, O), lambda i: (i, 0)),
        out_shape=jax.ShapeDtypeStruct((N, O), jnp.float32),
    )(numer, denom[:, :N].reshape(NC, nb, 1, bs), b_conv.reshape(1, H), W1,
      b1.reshape(1, H), W2, b2.reshape(1, O))


@jax.jit
def kernel(x, edge_index, W_src, W_dst, att_src, att_dst, b_conv, W1, b1, W2, b2):
    h_src, a_src, a_dst = _projections(x, W_src, W_dst, att_src, att_dst)

    src = edge_index[0].astype(jnp.int32)
    dst = edge_index[1].astype(jnp.int32)
    a_src_p = jnp.concatenate([a_src, jnp.zeros((N_PAD - N,), jnp.float32)])
    a_dst_p = jnp.concatenate([a_dst, jnp.zeros((N_PAD - N,), jnp.float32)])

    numer, denom = _edge_pass(src, dst, a_src_p, a_dst_p, h_src)

    return _mlp(numer, denom, b_conv, W1, b1, W2, b2)
